# baseline (speedup 1.0000x reference)
import jax
import jax.numpy as jnp
from jax import lax
from jax.experimental import pallas as pl
from jax.experimental.pallas import tpu as pltpu
from jax.experimental.pallas import tpu_sc as plsc

VOC = 1000000
EMB = 64
BATCH = 4096
SEQ = 200
NC, NS = 2, 16
J_PER_CORE = EMB // NC
MAX_ROWS = (SEQ + NS - 1) // NS
RING = 3


def kernel(tok_ids, word_emb):
    tab_t = word_emb.T
    tok_t = tok_ids.T

    mesh = plsc.VectorSubcoreMesh(core_axis_name="c", subcore_axis_name="s")

    @pl.kernel(
        out_type=jax.ShapeDtypeStruct((SEQ, EMB, BATCH), jnp.float32),
        mesh=mesh,
        scratch_types=[
            pltpu.VMEM_SHARED((VOC,), jnp.float32),
            pltpu.VMEM((MAX_ROWS * BATCH,), jnp.int32),
            pltpu.VMEM((RING * BATCH,), jnp.float32),
            pltpu.SemaphoreType.DMA,
            pltpu.SemaphoreType.DMA,
        ],
    )
    def sc_embed(tab_hbm, tok_hbm, out_hbm, row_sh, idx_v, dst_v, sem_g,
                 sem_o):
        cid = lax.axis_index("c")
        sid = lax.axis_index("s")
        nk = (SEQ - sid + NS - 1) // NS

        def idx_at(k):
            return idx_v.at[pl.ds(k * BATCH, BATCH)]

        def ring_g(g):
            return dst_v.at[pl.ds((g % RING) * BATCH, BATCH)]

        def out_g(g):
            return out_hbm.at[(g % nk) * NS + sid,
                              cid * J_PER_CORE + g // nk, :]

        def wait_out(g):
            pltpu.make_async_copy(ring_g(g), out_g(g), sem_o).wait()

        @pl.loop(0, MAX_ROWS)
        def _(k):
            @pl.when(k * NS + sid < SEQ)
            def _():
                pltpu.async_copy(tok_hbm.at[k * NS + sid], idx_at(k), sem_g)

        @pl.loop(0, MAX_ROWS)
        def _(k):
            @pl.when(k * NS + sid < SEQ)
            def _():
                pltpu.make_async_copy(
                    tok_hbm.at[k * NS + sid], idx_at(k), sem_g).wait()

        @pl.loop(0, J_PER_CORE)
        def _(j):
            jj = cid * J_PER_CORE + j

            @pl.when(sid == 0)
            def _():
                pltpu.sync_copy(tab_hbm.at[jj], row_sh)

            plsc.subcore_barrier()

            @pl.loop(0, MAX_ROWS)
            def _(k):
                @pl.when(k < nk)
                def _():
                    g = j * nk + k
                    pltpu.async_copy(row_sh.at[idx_at(k)], ring_g(g), sem_g)

            @pl.loop(0, MAX_ROWS)
            def _(k):
                @pl.when(k < nk)
                def _():
                    g = j * nk + k
                    pltpu.make_async_copy(
                        row_sh.at[idx_at(k)], ring_g(g), sem_g).wait()

            plsc.subcore_barrier()

    out = sc_embed(tab_t, tok_t)
    return jnp.transpose(out, (2, 0, 1))


# speedup vs baseline: 1.0055x; 1.0055x over previous
import jax
import jax.numpy as jnp
from jax import lax
from jax.experimental import pallas as pl
from jax.experimental.pallas import tpu as pltpu
from jax.experimental.pallas import tpu_sc as plsc

VOC = 1000000
EMB = 64
BATCH = 4096
SEQ = 200
NC, NS = 2, 16
J_PER_CORE = EMB // NC
HB = BATCH // 2
NK = SEQ * 2 // NS
RING = 3


def kernel(tok_ids, word_emb):
    tab_t = word_emb.T
    tok_t = tok_ids.T

    mesh = plsc.VectorSubcoreMesh(core_axis_name="c", subcore_axis_name="s")

    @pl.kernel(
        out_type=jax.ShapeDtypeStruct((SEQ, EMB, BATCH), jnp.float32),
        mesh=mesh,
        scratch_types=[
            pltpu.VMEM_SHARED((VOC,), jnp.float32),
            pltpu.VMEM((NK * HB,), jnp.int32),
            pltpu.VMEM((RING * HB,), jnp.float32),
            pltpu.SemaphoreType.DMA,
            pltpu.SemaphoreType.DMA,
        ],
    )
    def sc_embed(tab_hbm, tok_hbm, out_hbm, row_sh, idx_v, dst_v, sem_g,
                 sem_o):
        cid = lax.axis_index("c")
        sid = lax.axis_index("s")

        def idx_at(k):
            return idx_v.at[pl.ds(k * HB, HB)]

        def ring_g(g):
            return dst_v.at[pl.ds((g % RING) * HB, HB)]

        def out_g(g, jj):
            hr = (g % NK) * NS + sid
            return out_hbm.at[hr // 2, jj, pl.ds((hr % 2) * HB, HB)]

        def wait_out(g):
            pltpu.make_async_copy(
                ring_g(g), out_g(g, cid * J_PER_CORE + g // NK), sem_o).wait()

        @pl.loop(0, NK)
        def _(k):
            hr = k * NS + sid
            pltpu.async_copy(
                tok_hbm.at[hr // 2, pl.ds((hr % 2) * HB, HB)],
                idx_at(k), sem_g)

        @pl.loop(0, NK)
        def _(k):
            hr = k * NS + sid
            pltpu.make_async_copy(
                tok_hbm.at[hr // 2, pl.ds((hr % 2) * HB, HB)],
                idx_at(k), sem_g).wait()

        @pl.loop(0, J_PER_CORE)
        def _(j):
            jj = cid * J_PER_CORE + j

            @pl.when(sid == 0)
            def _():
                pltpu.sync_copy(tab_hbm.at[jj], row_sh)

            plsc.subcore_barrier()

            @pl.loop(0, NK)
            def _(k):
                g = j * NK + k

                @pl.when(g >= RING)
                def _():
                    wait_out(g - RING)

                pltpu.async_copy(row_sh.at[idx_at(k)], ring_g(g), sem_g)

                @pl.when(k >= 2)
                def _():
                    pltpu.make_async_copy(
                        row_sh.at[idx_at(k - 2)], ring_g(g - 2),
                        sem_g).wait()
                    pltpu.async_copy(ring_g(g - 2), out_g(g - 2, jj), sem_o)

            @pl.loop(NK - 2, NK)
            def _(k):
                g = j * NK + k
                pltpu.make_async_copy(
                    row_sh.at[idx_at(k)], ring_g(g), sem_g).wait()
                pltpu.async_copy(ring_g(g), out_g(g, jj), sem_o)

            plsc.subcore_barrier()

        @pl.loop(0, RING)
        def _(t):
            wait_out(J_PER_CORE * NK - RING + t)

    out = sc_embed(tab_t, tok_t)
    return jnp.transpose(out, (2, 0, 1))


# speedup vs baseline: 1.0242x; 1.0187x over previous
import jax
import jax.numpy as jnp
from jax import lax
from jax.experimental import pallas as pl
from jax.experimental.pallas import tpu as pltpu
from jax.experimental.pallas import tpu_sc as plsc

VOC = 1000000
EMB = 64
BATCH = 4096
SEQ = 200
NC, NS = 2, 16
J_PER_CORE = EMB // NC
HB = BATCH // 2
NK = SEQ * 2 // NS
RING = 3


def kernel(tok_ids, word_emb):
    tab_t = word_emb.T
    tok_t = tok_ids.T

    mesh = plsc.VectorSubcoreMesh(core_axis_name="c", subcore_axis_name="s")

    @pl.kernel(
        out_type=jax.ShapeDtypeStruct((SEQ, EMB, BATCH), jnp.float32),
        mesh=mesh,
        scratch_types=[
            pltpu.VMEM_SHARED((VOC,), jnp.float32),
            pltpu.VMEM((NK * HB,), jnp.int32),
            pltpu.VMEM((RING * HB,), jnp.float32),
            pltpu.SemaphoreType.DMA((3,)),
            pltpu.SemaphoreType.DMA((RING,)),
        ],
    )
    def sc_embed(tab_hbm, tok_hbm, out_hbm, row_sh, idx_v, dst_v, sem_g,
                 sem_o):
        cid = lax.axis_index("c")
        sid = lax.axis_index("s")

        def idx_at(k):
            return idx_v.at[pl.ds(k * HB, HB)]

        def sem_gk(k):
            return sem_g.at[k % 3]

        def ring_g(g):
            return dst_v.at[pl.ds((g % RING) * HB, HB)]

        def out_g(g, jj):
            hr = (g % NK) * NS + sid
            return out_hbm.at[hr // 2, jj, pl.ds((hr % 2) * HB, HB)]

        def wait_out(g):
            pltpu.make_async_copy(
                ring_g(g), out_g(g, cid * J_PER_CORE + g // NK),
                sem_o.at[g % RING]).wait()

        @pl.loop(0, NK)
        def _(k):
            hr = k * NS + sid
            pltpu.async_copy(
                tok_hbm.at[hr // 2, pl.ds((hr % 2) * HB, HB)],
                idx_at(k), sem_gk(k))

        @pl.loop(0, NK)
        def _(k):
            hr = k * NS + sid
            pltpu.make_async_copy(
                tok_hbm.at[hr // 2, pl.ds((hr % 2) * HB, HB)],
                idx_at(k), sem_gk(k)).wait()

        @pl.loop(0, J_PER_CORE)
        def _(j):
            jj = cid * J_PER_CORE + j

            @pl.when(sid == 0)
            def _():
                pltpu.sync_copy(tab_hbm.at[jj], row_sh)

            plsc.subcore_barrier()

            @pl.loop(0, NK)
            def _(k):
                g = j * NK + k

                @pl.when(g >= RING)
                def _():
                    wait_out(g - RING)

                pltpu.async_copy(row_sh.at[idx_at(k)], ring_g(g), sem_gk(k))

                @pl.when(k >= 2)
                def _():
                    pltpu.make_async_copy(
                        row_sh.at[idx_at(k - 2)], ring_g(g - 2),
                        sem_gk(k - 2)).wait()
                    pltpu.async_copy(ring_g(g - 2), out_g(g - 2, jj),
                                     sem_o.at[(g - 2) % RING])

            @pl.loop(NK - 2, NK)
            def _(k):
                g = j * NK + k
                pltpu.make_async_copy(
                    row_sh.at[idx_at(k)], ring_g(g), sem_gk(k)).wait()
                pltpu.async_copy(ring_g(g), out_g(g, jj),
                                 sem_o.at[g % RING])

            plsc.subcore_barrier()

        @pl.loop(0, RING)
        def _(t):
            wait_out(J_PER_CORE * NK - RING + t)

    out = sc_embed(tab_t, tok_t)
    return jnp.transpose(out, (2, 0, 1))


# speedup vs baseline: 1.0246x; 1.0004x over previous
import jax
import jax.numpy as jnp
from jax import lax
from jax.experimental import pallas as pl
from jax.experimental.pallas import tpu as pltpu
from jax.experimental.pallas import tpu_sc as plsc

VOC = 1000000
EMB = 64
BATCH = 4096
SEQ = 200
NC, NS = 2, 16
J_PER_CORE = EMB // NC
HB = BATCH // 2
NK = SEQ * 2 // NS
RING = 4


def kernel(tok_ids, word_emb):
    tab_t = word_emb.T
    tok_t = tok_ids.T

    mesh = plsc.VectorSubcoreMesh(core_axis_name="c", subcore_axis_name="s")

    @pl.kernel(
        out_type=jax.ShapeDtypeStruct((SEQ, EMB, BATCH), jnp.float32),
        mesh=mesh,
        scratch_types=[
            pltpu.VMEM_SHARED((VOC,), jnp.float32),
            pltpu.VMEM((NK * HB,), jnp.int32),
            pltpu.VMEM((RING * HB,), jnp.float32),
            pltpu.SemaphoreType.DMA((3,)),
            pltpu.SemaphoreType.DMA((RING,)),
        ],
    )
    def sc_embed(tab_hbm, tok_hbm, out_hbm, row_sh, idx_v, dst_v, sem_g,
                 sem_o):
        cid = lax.axis_index("c")
        sid = lax.axis_index("s")

        def idx_at(k):
            return idx_v.at[pl.ds(k * HB, HB)]

        def sem_gk(k):
            return sem_g.at[k % 3]

        def ring_g(g):
            return dst_v.at[pl.ds((g % RING) * HB, HB)]

        def out_g(g, jj):
            hr = (g % NK) * NS + sid
            return out_hbm.at[hr // 2, jj, pl.ds((hr % 2) * HB, HB)]

        def wait_out(g):
            pltpu.make_async_copy(
                ring_g(g), out_g(g, cid * J_PER_CORE + g // NK),
                sem_o.at[g % RING]).wait()

        @pl.loop(0, NK)
        def _(k):
            hr = k * NS + sid
            pltpu.async_copy(
                tok_hbm.at[hr // 2, pl.ds((hr % 2) * HB, HB)],
                idx_at(k), sem_gk(k))

        @pl.loop(0, NK)
        def _(k):
            hr = k * NS + sid
            pltpu.make_async_copy(
                tok_hbm.at[hr // 2, pl.ds((hr % 2) * HB, HB)],
                idx_at(k), sem_gk(k)).wait()

        @pl.loop(0, J_PER_CORE)
        def _(j):
            jj = cid * J_PER_CORE + j

            @pl.when(sid == 0)
            def _():
                pltpu.sync_copy(tab_hbm.at[jj], row_sh)

            plsc.subcore_barrier()

            @pl.loop(0, NK)
            def _(k):
                g = j * NK + k

                @pl.when(g >= RING)
                def _():
                    wait_out(g - RING)

                pltpu.async_copy(row_sh.at[idx_at(k)], ring_g(g), sem_gk(k))

                @pl.when(k >= 2)
                def _():
                    pltpu.make_async_copy(
                        row_sh.at[idx_at(k - 2)], ring_g(g - 2),
                        sem_gk(k - 2)).wait()
                    pltpu.async_copy(ring_g(g - 2), out_g(g - 2, jj),
                                     sem_o.at[(g - 2) % RING])

            @pl.loop(NK - 2, NK)
            def _(k):
                g = j * NK + k
                pltpu.make_async_copy(
                    row_sh.at[idx_at(k)], ring_g(g), sem_gk(k)).wait()
                pltpu.async_copy(ring_g(g), out_g(g, jj),
                                 sem_o.at[g % RING])

            plsc.subcore_barrier()

        @pl.loop(0, RING)
        def _(t):
            wait_out(J_PER_CORE * NK - RING + t)

    out = sc_embed(tab_t, tok_t)
    return jnp.transpose(out, (2, 0, 1))
